# baseline (speedup 1.0000x reference)
import functools

import jax
import jax.numpy as jnp
from jax import lax
from jax.experimental import pallas as pl
from jax.experimental.pallas import tpu as pltpu
from jax.experimental.pallas import tpu_sc as plsc

NN = 10000
EE = 320000
HH = 128
NGRAPH = 100
GP = 104
SCALES = [1, 2, 4, 8, 16, 32]

NC = 2
NS = 16
NW = NC * NS
C = 88
NBUF = 4
NCH = 116
EPT = NCH * C
E2 = NW * EPT
NGRP = NCH // NBUF
AGN = 10008
RPT = 632
RPT_LAST = NN - 15 * RPT

_f32 = jnp.float32
_mesh = plsc.VectorSubcoreMesh(
    core_axis_name="c", subcore_axis_name="s", num_cores=NC)


def _S(shape):
    return jax.ShapeDtypeStruct(shape, _f32)


def _tile_rows(body_fn, s):
    row0 = pl.multiple_of(s * RPT, 8)

    @pl.when(s < 15)
    def _():
        body_fn(row0, RPT)

    @pl.when(s == 15)
    def _():
        body_fn(row0, RPT_LAST)


@functools.partial(
    pl.kernel,
    out_type=(_S((NN, HH)), _S((NN, HH))),
    mesh=_mesh,
    scratch_types=[
        pltpu.VMEM((NBUF, 1, C), jnp.int32),
        pltpu.VMEM((NBUF, 1, C), jnp.int32),
        pltpu.VMEM((NBUF, C, HH), _f32),
        pltpu.VMEM_SHARED((AGN, HH), _f32),
        pltpu.SemaphoreType.DMA((NBUF,)),
        pltpu.SemaphoreType.DMA((NBUF,)),
        pltpu.SemaphoreType.DMA((NBUF,)),
        pltpu.SemaphoreType.DMA((NBUF,)),
    ],
)
def _scatter_kernel(hd_hbm, src_hbm, dst_hbm, zeros_hbm, agg0_hbm, agg1_hbm,
                    isrc_v, idst_v, rows_v, agg_sh, isem, dsem, gsem, ssem):
    cc = lax.axis_index("c")
    s = lax.axis_index("s")
    w = cc * NS + s

    def zero(row0, nrows):
        pltpu.sync_copy(zeros_hbm.at[pl.ds(0, nrows)],
                        agg_sh.at[pl.ds(row0, nrows)])

    _tile_rows(zero, s)

    @pl.when(s == 15)
    def _():
        pltpu.sync_copy(zeros_hbm.at[pl.ds(0, AGN - NN)],
                        agg_sh.at[pl.ds(NN, AGN - NN)])

    plsc.subcore_barrier()

    for b in range(NBUF):
        pltpu.async_copy(src_hbm.at[w, b], isrc_v.at[b], isem.at[b])

    def grp(g, carry):
        for b in range(NBUF):
            j = g * NBUF + b

            @pl.when(g > 0)
            def _():
                pltpu.make_async_copy(
                    rows_v.at[b], agg_sh.at[idst_v.at[b, 0]],
                    ssem.at[b]).wait()

            pltpu.async_copy(dst_hbm.at[w, j], idst_v.at[b], dsem.at[b])
            pltpu.make_async_copy(
                src_hbm.at[w, j], isrc_v.at[b], isem.at[b]).wait()
            pltpu.async_copy(
                hd_hbm.at[isrc_v.at[b, 0]], rows_v.at[b], gsem.at[b])
        for b in range(NBUF):
            j = g * NBUF + b
            pltpu.make_async_copy(
                hd_hbm.at[isrc_v.at[b, 0]], rows_v.at[b], gsem.at[b]).wait()

            @pl.when(g < NGRP - 1)
            def _():
                pltpu.async_copy(
                    src_hbm.at[w, j + NBUF], isrc_v.at[b], isem.at[b])

            pltpu.make_async_copy(
                dst_hbm.at[w, j], idst_v.at[b], dsem.at[b]).wait()
            pltpu.async_copy(
                rows_v.at[b], agg_sh.at[idst_v.at[b, 0]], ssem.at[b],
                add=True)
        return carry

    lax.fori_loop(0, NGRP, grp, 0)
    for b in range(NBUF):
        pltpu.make_async_copy(
            rows_v.at[b], agg_sh.at[idst_v.at[b, 0]], ssem.at[b]).wait()
    plsc.subcore_barrier()

    def export0(row0, nrows):
        pltpu.sync_copy(agg_sh.at[pl.ds(row0, nrows)],
                        agg0_hbm.at[pl.ds(row0, nrows)])

    def export1(row0, nrows):
        pltpu.sync_copy(agg_sh.at[pl.ds(row0, nrows)],
                        agg1_hbm.at[pl.ds(row0, nrows)])

    @pl.when(cc == 0)
    def _():
        _tile_rows(export0, s)

    @pl.when(cc == 1)
    def _():
        _tile_rows(export1, s)


RB = 400
RBU = 1000


def _init_body(x, w, b, d0, d1, bt, h_ref, hd_ref, dv_ref, cnt_ref):
    i = pl.program_id(0)
    h = jnp.dot(x[...], w[...], preferred_element_type=_f32) + b[...]
    dinv = 1.0 / jnp.maximum(d0[...][:, :1] + d1[...][:, :1], 1.0)
    h_ref[...] = h
    hd_ref[...] = h * dinv
    dv_ref[...] = dinv * jnp.ones((1, 16), _f32)
    oh = (bt[...] == lax.broadcasted_iota(jnp.int32, (1, GP), 1)).astype(_f32)
    cnt = jnp.sum(oh, axis=0)[:, None] * jnp.ones((1, HH), _f32)

    @pl.when(i == 0)
    def _():
        cnt_ref[...] = cnt

    @pl.when(i != 0)
    def _():
        cnt_ref[...] += cnt


_init_call = pl.pallas_call(
    _init_body,
    grid=(NN // RB,),
    in_specs=[
        pl.BlockSpec((RB, HH), lambda i: (i, 0)),
        pl.BlockSpec((HH, HH), lambda i: (0, 0)),
        pl.BlockSpec((1, HH), lambda i: (0, 0)),
        pl.BlockSpec((RB, HH), lambda i: (i, 0)),
        pl.BlockSpec((RB, HH), lambda i: (i, 0)),
        pl.BlockSpec((RB, 1), lambda i: (i, 0)),
    ],
    out_specs=[
        pl.BlockSpec((RB, HH), lambda i: (i, 0)),
        pl.BlockSpec((RB, HH), lambda i: (i, 0)),
        pl.BlockSpec((RB, 16), lambda i: (i, 0)),
        pl.BlockSpec((GP, HH), lambda i: (0, 0)),
    ],
    out_shape=[_S((NN, HH)), _S((NN, HH)), _S((NN, 16)), _S((GP, HH))],
)


def _update_body(h, a0, a1, dv, hn_ref, hdn_ref):
    hn = 0.5 * (h[...] + (a0[...] + a1[...]))
    hn_ref[...] = hn
    hdn_ref[...] = hn * dv[...][:, :1]


_update_call = pl.pallas_call(
    _update_body,
    grid=(NN // RBU,),
    in_specs=[
        pl.BlockSpec((RBU, HH), lambda i: (i, 0)),
        pl.BlockSpec((RBU, HH), lambda i: (i, 0)),
        pl.BlockSpec((RBU, HH), lambda i: (i, 0)),
        pl.BlockSpec((RBU, 16), lambda i: (i, 0)),
    ],
    out_specs=[
        pl.BlockSpec((RBU, HH), lambda i: (i, 0)),
        pl.BlockSpec((RBU, HH), lambda i: (i, 0)),
    ],
    out_shape=[_S((NN, HH)), _S((NN, HH))],
)


def _mlp_core(ps, w1, b1, w2, b2):
    acc = b1
    for j in range(6):
        wav = jnp.maximum(ps[j] - ps[j + 1], 0.0)
        acc = acc + jnp.dot(wav, w1[j], preferred_element_type=_f32)
    z = jnp.maximum(acc, 0.0)
    return jnp.dot(z, w2, preferred_element_type=_f32) + b2


def _mlp1_body(p0, p1, p2, p3, p4, p5, p6, w1, b1, w2, b2, dv,
               h_ref, hd_ref):
    ps = [p0[...], p1[...], p2[...], p3[...], p4[...], p5[...], p6[...]]
    out = _mlp_core(ps, w1[...], b1[...], w2[...], b2[...])
    h_ref[...] = out
    hd_ref[...] = out * dv[...][:, :1]


_p_spec = pl.BlockSpec((RB, HH), lambda i: (i, 0))
_w_specs = [
    pl.BlockSpec((6, HH, HH), lambda i: (0, 0, 0)),
    pl.BlockSpec((1, HH), lambda i: (0, 0)),
    pl.BlockSpec((HH, HH), lambda i: (0, 0)),
    pl.BlockSpec((1, HH), lambda i: (0, 0)),
]

_mlp1_call = pl.pallas_call(
    _mlp1_body,
    grid=(NN // RB,),
    in_specs=[_p_spec] * 7 + _w_specs + [pl.BlockSpec((RB, 16), lambda i: (i, 0))],
    out_specs=[_p_spec, _p_spec],
    out_shape=[_S((NN, HH)), _S((NN, HH))],
)


def _mlp2_body(p0, p1, p2, p3, p4, p5, p6, w1, b1, w2, b2, bt, sum_ref):
    i = pl.program_id(0)
    ps = [p0[...], p1[...], p2[...], p3[...], p4[...], p5[...], p6[...]]
    out = _mlp_core(ps, w1[...], b1[...], w2[...], b2[...])
    oh = (bt[...] == lax.broadcasted_iota(jnp.int32, (1, GP), 1)).astype(_f32)
    part = lax.dot_general(oh, out, (((0,), (0,)), ((), ())),
                           preferred_element_type=_f32)

    @pl.when(i == 0)
    def _():
        sum_ref[...] = part

    @pl.when(i != 0)
    def _():
        sum_ref[...] += part


_mlp2_call = pl.pallas_call(
    _mlp2_body,
    grid=(NN // RB,),
    in_specs=[_p_spec] * 7 + _w_specs + [pl.BlockSpec((RB, 1), lambda i: (i, 0))],
    out_specs=pl.BlockSpec((GP, HH), lambda i: (0, 0)),
    out_shape=_S((GP, HH)),
)


def _final_body(sums, cnts, w, b, out_ref):
    pooled = sums[...] / jnp.maximum(cnts[...], 1.0)
    r = jnp.dot(pooled, w[...], preferred_element_type=_f32) + b[...]
    out_ref[...] = r[:NGRAPH]


_final_call = pl.pallas_call(
    _final_body,
    grid=(1,),
    in_specs=[
        pl.BlockSpec((GP, HH), lambda i: (0, 0)),
        pl.BlockSpec((GP, HH), lambda i: (0, 0)),
        pl.BlockSpec((HH, HH), lambda i: (0, 0)),
        pl.BlockSpec((1, HH), lambda i: (0, 0)),
    ],
    out_specs=pl.BlockSpec((NGRAPH, HH), lambda i: (0, 0)),
    out_shape=_S((NGRAPH, HH)),
)


def kernel(x, edge_index, batch, W_in, b_in, W1_0, b1_0, W2_0, b2_0,
           W1_1, b1_1, W2_1, b2_1, W_lin, b_lin):
    src = edge_index[0]
    dst = edge_index[1]
    padG = jnp.zeros((E2 - EE,), jnp.int32)
    padD = jnp.full((E2 - EE,), NN, jnp.int32)
    srcG = jnp.concatenate([src, padG]).reshape(NW, NCH, 1, C)
    srcD = jnp.concatenate([src, padD]).reshape(NW, NCH, 1, C)
    dstP = jnp.concatenate([dst, padD]).reshape(NW, NCH, 1, C)
    ones128 = jnp.ones((NN, HH), _f32)
    zeros128 = jnp.zeros((RPT, HH), _f32)
    batch2 = batch.reshape(NN, 1)

    deg0, deg1 = _scatter_kernel(ones128, srcG, srcD, zeros128)
    h, hd, dinv16, counts = _init_call(
        x, W_in, b_in.reshape(1, HH), deg0, deg1, batch2)

    sums = None
    for layer, (W1, b1, W2, b2) in enumerate(
            ((W1_0, b1_0, W2_0, b2_0), (W1_1, b1_1, W2_1, b2_1))):
        powers = [h]
        step = 0
        for tgt in SCALES:
            while step < tgt:
                a0, a1 = _scatter_kernel(hd, srcG, dstP, zeros128)
                h, hd = _update_call(h, a0, a1, dinv16)
                step += 1
            powers.append(h)
        w1r = W1.reshape(6, HH, HH)
        if layer == 0:
            h, hd = _mlp1_call(*powers, w1r, b1.reshape(1, HH),
                               W2, b2.reshape(1, HH), dinv16)
        else:
            sums = _mlp2_call(*powers, w1r, b1.reshape(1, HH),
                              W2, b2.reshape(1, HH), batch2)

    return _final_call(sums, counts, W_lin, b_lin.reshape(1, HH))


# speedup vs baseline: 3.2018x; 3.2018x over previous
import functools

import jax
import jax.numpy as jnp
from jax import lax
from jax.experimental import pallas as pl
from jax.experimental.pallas import tpu as pltpu
from jax.experimental.pallas import tpu_sc as plsc

NN = 10000
EE = 320000
HH = 128
NGRAPH = 100
GP = 104
SCALES = [1, 2, 4, 8, 16, 32]

NC = 2
NS = 16
NW = NC * NS
C = 112
NBUF = 3
NCH = 90
EPT = NCH * C
E2 = NW * EPT
NGRP = NCH // NBUF
AGN = 10504
RPT = 632
RPT_LAST = NN - 15 * RPT

_f32 = jnp.float32
_mesh = plsc.VectorSubcoreMesh(
    core_axis_name="c", subcore_axis_name="s", num_cores=NC)


def _S(shape):
    return jax.ShapeDtypeStruct(shape, _f32)


def _tile_rows(body_fn, s):
    row0 = pl.multiple_of(s * RPT, 8)

    @pl.when(s < 15)
    def _():
        body_fn(row0, RPT)

    @pl.when(s == 15)
    def _():
        body_fn(row0, RPT_LAST)


@functools.partial(
    pl.kernel,
    out_type=(_S((NN, HH)), _S((NN, HH))),
    mesh=_mesh,
    scratch_types=[
        pltpu.VMEM((NBUF, 1, C), jnp.int32),
        pltpu.VMEM((NBUF, 1, C), jnp.int32),
        pltpu.VMEM((NBUF, C, HH), _f32),
        pltpu.VMEM_SHARED((AGN, HH), _f32),
        pltpu.SemaphoreType.DMA((NBUF,)),
        pltpu.SemaphoreType.DMA((NBUF,)),
        pltpu.SemaphoreType.DMA((NBUF,)),
        pltpu.SemaphoreType.DMA((NBUF,)),
    ],
)
def _scatter_kernel(hd_hbm, src_hbm, dst_hbm, zeros_hbm, agg0_hbm, agg1_hbm,
                    isrc_v, idst_v, rows_v, agg_sh, isem, dsem, gsem, ssem):
    cc = lax.axis_index("c")
    s = lax.axis_index("s")
    w = cc * NS + s

    def zero(row0, nrows):
        pltpu.sync_copy(zeros_hbm.at[pl.ds(0, nrows)],
                        agg_sh.at[pl.ds(row0, nrows)])

    _tile_rows(zero, s)

    @pl.when(s == 15)
    def _():
        pltpu.sync_copy(zeros_hbm.at[pl.ds(0, AGN - NN)],
                        agg_sh.at[pl.ds(NN, AGN - NN)])

    plsc.subcore_barrier()

    for b in range(NBUF):
        pltpu.async_copy(src_hbm.at[w, b], isrc_v.at[b], isem.at[b])

    def grp(g, carry):
        for b in range(NBUF):
            j = g * NBUF + b

            @pl.when(g > 0)
            def _():
                pltpu.make_async_copy(
                    rows_v.at[b], agg_sh.at[idst_v.at[b, 0]],
                    ssem.at[b]).wait()

            pltpu.async_copy(dst_hbm.at[w, j], idst_v.at[b], dsem.at[b])
            pltpu.make_async_copy(
                src_hbm.at[w, j], isrc_v.at[b], isem.at[b]).wait()
            pltpu.async_copy(
                hd_hbm.at[isrc_v.at[b, 0]], rows_v.at[b], gsem.at[b])
        for b in range(NBUF):
            j = g * NBUF + b
            pltpu.make_async_copy(
                hd_hbm.at[isrc_v.at[b, 0]], rows_v.at[b], gsem.at[b]).wait()

            @pl.when(g < NGRP - 1)
            def _():
                pltpu.async_copy(
                    src_hbm.at[w, j + NBUF], isrc_v.at[b], isem.at[b])

            pltpu.make_async_copy(
                dst_hbm.at[w, j], idst_v.at[b], dsem.at[b]).wait()
            pltpu.async_copy(
                rows_v.at[b], agg_sh.at[idst_v.at[b, 0]], ssem.at[b],
                add=True)
        return carry

    lax.fori_loop(0, NGRP, grp, 0)
    for b in range(NBUF):
        pltpu.make_async_copy(
            rows_v.at[b], agg_sh.at[idst_v.at[b, 0]], ssem.at[b]).wait()
    plsc.subcore_barrier()

    def export0(row0, nrows):
        pltpu.sync_copy(agg_sh.at[pl.ds(row0, nrows)],
                        agg0_hbm.at[pl.ds(row0, nrows)])

    def export1(row0, nrows):
        pltpu.sync_copy(agg_sh.at[pl.ds(row0, nrows)],
                        agg1_hbm.at[pl.ds(row0, nrows)])

    @pl.when(cc == 0)
    def _():
        _tile_rows(export0, s)

    @pl.when(cc == 1)
    def _():
        _tile_rows(export1, s)


RB = 400
RBU = 1000


def _init_body(x, w, b, d0, d1, bt, h_ref, hd_ref, dv_ref, cnt_ref):
    i = pl.program_id(0)
    h = jnp.dot(x[...], w[...], preferred_element_type=_f32) + b[...]
    dinv = 1.0 / jnp.maximum(d0[...][:, :1] + d1[...][:, :1], 1.0)
    h_ref[...] = h
    hd_ref[...] = h * dinv
    dv_ref[...] = dinv * jnp.ones((1, 16), _f32)
    oh = (bt[...] == lax.broadcasted_iota(jnp.int32, (1, GP), 1)).astype(_f32)
    cnt = jnp.sum(oh, axis=0)[:, None] * jnp.ones((1, HH), _f32)

    @pl.when(i == 0)
    def _():
        cnt_ref[...] = cnt

    @pl.when(i != 0)
    def _():
        cnt_ref[...] += cnt


_init_call = pl.pallas_call(
    _init_body,
    grid=(NN // RB,),
    in_specs=[
        pl.BlockSpec((RB, HH), lambda i: (i, 0)),
        pl.BlockSpec((HH, HH), lambda i: (0, 0)),
        pl.BlockSpec((1, HH), lambda i: (0, 0)),
        pl.BlockSpec((RB, HH), lambda i: (i, 0)),
        pl.BlockSpec((RB, HH), lambda i: (i, 0)),
        pl.BlockSpec((RB, 1), lambda i: (i, 0)),
    ],
    out_specs=[
        pl.BlockSpec((RB, HH), lambda i: (i, 0)),
        pl.BlockSpec((RB, HH), lambda i: (i, 0)),
        pl.BlockSpec((RB, 16), lambda i: (i, 0)),
        pl.BlockSpec((GP, HH), lambda i: (0, 0)),
    ],
    out_shape=[_S((NN, HH)), _S((NN, HH)), _S((NN, 16)), _S((GP, HH))],
)


def _update_body(h, a0, a1, dv, hn_ref, hdn_ref):
    hn = 0.5 * (h[...] + (a0[...] + a1[...]))
    hn_ref[...] = hn
    hdn_ref[...] = hn * dv[...][:, :1]


_update_call = pl.pallas_call(
    _update_body,
    grid=(NN // RBU,),
    in_specs=[
        pl.BlockSpec((RBU, HH), lambda i: (i, 0)),
        pl.BlockSpec((RBU, HH), lambda i: (i, 0)),
        pl.BlockSpec((RBU, HH), lambda i: (i, 0)),
        pl.BlockSpec((RBU, 16), lambda i: (i, 0)),
    ],
    out_specs=[
        pl.BlockSpec((RBU, HH), lambda i: (i, 0)),
        pl.BlockSpec((RBU, HH), lambda i: (i, 0)),
    ],
    out_shape=[_S((NN, HH)), _S((NN, HH))],
)


def _mlp_core(ps, w1, b1, w2, b2):
    acc = b1
    for j in range(6):
        wav = jnp.maximum(ps[j] - ps[j + 1], 0.0)
        acc = acc + jnp.dot(wav, w1[j], preferred_element_type=_f32)
    z = jnp.maximum(acc, 0.0)
    return jnp.dot(z, w2, preferred_element_type=_f32) + b2


def _mlp1_body(p0, p1, p2, p3, p4, p5, p6, w1, b1, w2, b2, dv,
               h_ref, hd_ref):
    ps = [p0[...], p1[...], p2[...], p3[...], p4[...], p5[...], p6[...]]
    out = _mlp_core(ps, w1[...], b1[...], w2[...], b2[...])
    h_ref[...] = out
    hd_ref[...] = out * dv[...][:, :1]


_p_spec = pl.BlockSpec((RB, HH), lambda i: (i, 0))
_w_specs = [
    pl.BlockSpec((6, HH, HH), lambda i: (0, 0, 0)),
    pl.BlockSpec((1, HH), lambda i: (0, 0)),
    pl.BlockSpec((HH, HH), lambda i: (0, 0)),
    pl.BlockSpec((1, HH), lambda i: (0, 0)),
]

_mlp1_call = pl.pallas_call(
    _mlp1_body,
    grid=(NN // RB,),
    in_specs=[_p_spec] * 7 + _w_specs + [pl.BlockSpec((RB, 16), lambda i: (i, 0))],
    out_specs=[_p_spec, _p_spec],
    out_shape=[_S((NN, HH)), _S((NN, HH))],
)


def _mlp2_body(p0, p1, p2, p3, p4, p5, p6, w1, b1, w2, b2, bt, sum_ref):
    i = pl.program_id(0)
    ps = [p0[...], p1[...], p2[...], p3[...], p4[...], p5[...], p6[...]]
    out = _mlp_core(ps, w1[...], b1[...], w2[...], b2[...])
    oh = (bt[...] == lax.broadcasted_iota(jnp.int32, (1, GP), 1)).astype(_f32)
    part = lax.dot_general(oh, out, (((0,), (0,)), ((), ())),
                           preferred_element_type=_f32)

    @pl.when(i == 0)
    def _():
        sum_ref[...] = part

    @pl.when(i != 0)
    def _():
        sum_ref[...] += part


_mlp2_call = pl.pallas_call(
    _mlp2_body,
    grid=(NN // RB,),
    in_specs=[_p_spec] * 7 + _w_specs + [pl.BlockSpec((RB, 1), lambda i: (i, 0))],
    out_specs=pl.BlockSpec((GP, HH), lambda i: (0, 0)),
    out_shape=_S((GP, HH)),
)


def _final_body(sums, cnts, w, b, out_ref):
    pooled = sums[...] / jnp.maximum(cnts[...], 1.0)
    r = jnp.dot(pooled, w[...], preferred_element_type=_f32) + b[...]
    out_ref[...] = r[:NGRAPH]


_final_call = pl.pallas_call(
    _final_body,
    grid=(1,),
    in_specs=[
        pl.BlockSpec((GP, HH), lambda i: (0, 0)),
        pl.BlockSpec((GP, HH), lambda i: (0, 0)),
        pl.BlockSpec((HH, HH), lambda i: (0, 0)),
        pl.BlockSpec((1, HH), lambda i: (0, 0)),
    ],
    out_specs=pl.BlockSpec((NGRAPH, HH), lambda i: (0, 0)),
    out_shape=_S((NGRAPH, HH)),
)


def kernel(x, edge_index, batch, W_in, b_in, W1_0, b1_0, W2_0, b2_0,
           W1_1, b1_1, W2_1, b2_1, W_lin, b_lin):
    src = edge_index[0]
    dst = edge_index[1]
    npad = E2 - EE
    padG = jnp.arange(npad, dtype=jnp.int32) % NN
    padD = NN + (jnp.arange(npad, dtype=jnp.int32) % (AGN - NN))
    srcG = jnp.concatenate([src, padG]).reshape(NW, NCH, 1, C)
    srcD = jnp.concatenate([src, padD]).reshape(NW, NCH, 1, C)
    dstP = jnp.concatenate([dst, padD]).reshape(NW, NCH, 1, C)
    ones128 = jnp.ones((NN, HH), _f32)
    zeros128 = jnp.zeros((RPT, HH), _f32)
    batch2 = batch.reshape(NN, 1)

    deg0, deg1 = _scatter_kernel(ones128, srcG, srcD, zeros128)
    h, hd, dinv16, counts = _init_call(
        x, W_in, b_in.reshape(1, HH), deg0, deg1, batch2)

    sums = None
    for layer, (W1, b1, W2, b2) in enumerate(
            ((W1_0, b1_0, W2_0, b2_0), (W1_1, b1_1, W2_1, b2_1))):
        powers = [h]
        step = 0
        for tgt in SCALES:
            while step < tgt:
                a0, a1 = _scatter_kernel(hd, srcG, dstP, zeros128)
                h, hd = _update_call(h, a0, a1, dinv16)
                step += 1
            powers.append(h)
        w1r = W1.reshape(6, HH, HH)
        if layer == 0:
            h, hd = _mlp1_call(*powers, w1r, b1.reshape(1, HH),
                               W2, b2.reshape(1, HH), dinv16)
        else:
            sums = _mlp2_call(*powers, w1r, b1.reshape(1, HH),
                              W2, b2.reshape(1, HH), batch2)

    return _final_call(sums, counts, W_lin, b_lin.reshape(1, HH))


# speedup vs baseline: 3.3876x; 1.0580x over previous
import functools

import jax
import jax.numpy as jnp
from jax import lax
from jax.experimental import pallas as pl
from jax.experimental.pallas import tpu as pltpu
from jax.experimental.pallas import tpu_sc as plsc

NN = 10000
EE = 320000
HH = 128
NGRAPH = 100
GP = 104
SCALES = [1, 2, 4, 8, 16, 32]

NC = 2
NS = 16
NW = NC * NS
C = 88
NBUF = 4
NCH = 116
EPT = NCH * C
E2 = NW * EPT
NGRP = NCH // NBUF
AGN = 10504
RPT = 632
RPT_LAST = NN - 15 * RPT

_f32 = jnp.float32
_mesh = plsc.VectorSubcoreMesh(
    core_axis_name="c", subcore_axis_name="s", num_cores=NC)


def _S(shape):
    return jax.ShapeDtypeStruct(shape, _f32)


def _tile_rows(body_fn, s):
    row0 = pl.multiple_of(s * RPT, 8)

    @pl.when(s < 15)
    def _():
        body_fn(row0, RPT)

    @pl.when(s == 15)
    def _():
        body_fn(row0, RPT_LAST)


@functools.partial(
    pl.kernel,
    out_type=(_S((NN, HH)), _S((NN, HH))),
    mesh=_mesh,
    scratch_types=[
        pltpu.VMEM((NBUF, 1, C), jnp.int32),
        pltpu.VMEM((NBUF, 1, C), jnp.int32),
        pltpu.VMEM((NBUF, C, HH), _f32),
        pltpu.VMEM_SHARED((AGN, HH), _f32),
        pltpu.SemaphoreType.DMA((NBUF,)),
        pltpu.SemaphoreType.DMA((NBUF,)),
        pltpu.SemaphoreType.DMA((NBUF,)),
        pltpu.SemaphoreType.DMA((NBUF,)),
    ],
)
def _scatter_kernel(hd_hbm, src_hbm, dst_hbm, zeros_hbm, agg0_hbm, agg1_hbm,
                    isrc_v, idst_v, rows_v, agg_sh, isem, dsem, gsem, ssem):
    cc = lax.axis_index("c")
    s = lax.axis_index("s")
    w = cc * NS + s

    def zero(row0, nrows):
        pltpu.sync_copy(zeros_hbm.at[pl.ds(0, nrows)],
                        agg_sh.at[pl.ds(row0, nrows)])

    _tile_rows(zero, s)

    @pl.when(s == 15)
    def _():
        pltpu.sync_copy(zeros_hbm.at[pl.ds(0, AGN - NN)],
                        agg_sh.at[pl.ds(NN, AGN - NN)])

    plsc.subcore_barrier()

    for b in range(NBUF):
        pltpu.async_copy(src_hbm.at[w, b], isrc_v.at[b], isem.at[b])

    def grp(g, carry):
        for b in range(NBUF):
            j = g * NBUF + b

            @pl.when(g > 0)
            def _():
                pltpu.make_async_copy(
                    rows_v.at[b], agg_sh.at[idst_v.at[b, 0]],
                    ssem.at[b]).wait()

            pltpu.async_copy(dst_hbm.at[w, j], idst_v.at[b], dsem.at[b])
            pltpu.make_async_copy(
                src_hbm.at[w, j], isrc_v.at[b], isem.at[b]).wait()
            pltpu.async_copy(
                hd_hbm.at[isrc_v.at[b, 0]], rows_v.at[b], gsem.at[b])
        for b in range(NBUF):
            j = g * NBUF + b
            pltpu.make_async_copy(
                hd_hbm.at[isrc_v.at[b, 0]], rows_v.at[b], gsem.at[b]).wait()

            @pl.when(g < NGRP - 1)
            def _():
                pltpu.async_copy(
                    src_hbm.at[w, j + NBUF], isrc_v.at[b], isem.at[b])

            pltpu.make_async_copy(
                dst_hbm.at[w, j], idst_v.at[b], dsem.at[b]).wait()
            pltpu.async_copy(
                rows_v.at[b], agg_sh.at[idst_v.at[b, 0]], ssem.at[b],
                add=True)
        return carry

    lax.fori_loop(0, NGRP, grp, 0)
    for b in range(NBUF):
        pltpu.make_async_copy(
            rows_v.at[b], agg_sh.at[idst_v.at[b, 0]], ssem.at[b]).wait()
    plsc.subcore_barrier()

    def export0(row0, nrows):
        pltpu.sync_copy(agg_sh.at[pl.ds(row0, nrows)],
                        agg0_hbm.at[pl.ds(row0, nrows)])

    def export1(row0, nrows):
        pltpu.sync_copy(agg_sh.at[pl.ds(row0, nrows)],
                        agg1_hbm.at[pl.ds(row0, nrows)])

    @pl.when(cc == 0)
    def _():
        _tile_rows(export0, s)

    @pl.when(cc == 1)
    def _():
        _tile_rows(export1, s)


RB = 400
RBU = 1000


def _init_body(x, w, b, d0, d1, bt, h_ref, hd_ref, dv_ref, cnt_ref):
    i = pl.program_id(0)
    h = jnp.dot(x[...], w[...], preferred_element_type=_f32) + b[...]
    dinv = 1.0 / jnp.maximum(d0[...][:, :1] + d1[...][:, :1], 1.0)
    h_ref[...] = h
    hd_ref[...] = h * dinv
    dv_ref[...] = dinv * jnp.ones((1, 16), _f32)
    oh = (bt[...] == lax.broadcasted_iota(jnp.int32, (1, GP), 1)).astype(_f32)
    cnt = jnp.sum(oh, axis=0)[:, None] * jnp.ones((1, HH), _f32)

    @pl.when(i == 0)
    def _():
        cnt_ref[...] = cnt

    @pl.when(i != 0)
    def _():
        cnt_ref[...] += cnt


_init_call = pl.pallas_call(
    _init_body,
    grid=(NN // RB,),
    in_specs=[
        pl.BlockSpec((RB, HH), lambda i: (i, 0)),
        pl.BlockSpec((HH, HH), lambda i: (0, 0)),
        pl.BlockSpec((1, HH), lambda i: (0, 0)),
        pl.BlockSpec((RB, HH), lambda i: (i, 0)),
        pl.BlockSpec((RB, HH), lambda i: (i, 0)),
        pl.BlockSpec((RB, 1), lambda i: (i, 0)),
    ],
    out_specs=[
        pl.BlockSpec((RB, HH), lambda i: (i, 0)),
        pl.BlockSpec((RB, HH), lambda i: (i, 0)),
        pl.BlockSpec((RB, 16), lambda i: (i, 0)),
        pl.BlockSpec((GP, HH), lambda i: (0, 0)),
    ],
    out_shape=[_S((NN, HH)), _S((NN, HH)), _S((NN, 16)), _S((GP, HH))],
)


def _update_body(h, a0, a1, dv, hn_ref, hdn_ref):
    hn = 0.5 * (h[...] + (a0[...] + a1[...]))
    hn_ref[...] = hn
    hdn_ref[...] = hn * dv[...][:, :1]


_update_call = pl.pallas_call(
    _update_body,
    grid=(NN // RBU,),
    in_specs=[
        pl.BlockSpec((RBU, HH), lambda i: (i, 0)),
        pl.BlockSpec((RBU, HH), lambda i: (i, 0)),
        pl.BlockSpec((RBU, HH), lambda i: (i, 0)),
        pl.BlockSpec((RBU, 16), lambda i: (i, 0)),
    ],
    out_specs=[
        pl.BlockSpec((RBU, HH), lambda i: (i, 0)),
        pl.BlockSpec((RBU, HH), lambda i: (i, 0)),
    ],
    out_shape=[_S((NN, HH)), _S((NN, HH))],
)


def _mlp_core(ps, w1, b1, w2, b2):
    acc = b1
    for j in range(6):
        wav = jnp.maximum(ps[j] - ps[j + 1], 0.0)
        acc = acc + jnp.dot(wav, w1[j], preferred_element_type=_f32)
    z = jnp.maximum(acc, 0.0)
    return jnp.dot(z, w2, preferred_element_type=_f32) + b2


def _mlp1_body(p0, p1, p2, p3, p4, p5, p6, w1, b1, w2, b2, dv,
               h_ref, hd_ref):
    ps = [p0[...], p1[...], p2[...], p3[...], p4[...], p5[...], p6[...]]
    out = _mlp_core(ps, w1[...], b1[...], w2[...], b2[...])
    h_ref[...] = out
    hd_ref[...] = out * dv[...][:, :1]


_p_spec = pl.BlockSpec((RB, HH), lambda i: (i, 0))
_w_specs = [
    pl.BlockSpec((6, HH, HH), lambda i: (0, 0, 0)),
    pl.BlockSpec((1, HH), lambda i: (0, 0)),
    pl.BlockSpec((HH, HH), lambda i: (0, 0)),
    pl.BlockSpec((1, HH), lambda i: (0, 0)),
]

_mlp1_call = pl.pallas_call(
    _mlp1_body,
    grid=(NN // RB,),
    in_specs=[_p_spec] * 7 + _w_specs + [pl.BlockSpec((RB, 16), lambda i: (i, 0))],
    out_specs=[_p_spec, _p_spec],
    out_shape=[_S((NN, HH)), _S((NN, HH))],
)


def _mlp2_body(p0, p1, p2, p3, p4, p5, p6, w1, b1, w2, b2, bt, sum_ref):
    i = pl.program_id(0)
    ps = [p0[...], p1[...], p2[...], p3[...], p4[...], p5[...], p6[...]]
    out = _mlp_core(ps, w1[...], b1[...], w2[...], b2[...])
    oh = (bt[...] == lax.broadcasted_iota(jnp.int32, (1, GP), 1)).astype(_f32)
    part = lax.dot_general(oh, out, (((0,), (0,)), ((), ())),
                           preferred_element_type=_f32)

    @pl.when(i == 0)
    def _():
        sum_ref[...] = part

    @pl.when(i != 0)
    def _():
        sum_ref[...] += part


_mlp2_call = pl.pallas_call(
    _mlp2_body,
    grid=(NN // RB,),
    in_specs=[_p_spec] * 7 + _w_specs + [pl.BlockSpec((RB, 1), lambda i: (i, 0))],
    out_specs=pl.BlockSpec((GP, HH), lambda i: (0, 0)),
    out_shape=_S((GP, HH)),
)


def _final_body(sums, cnts, w, b, out_ref):
    pooled = sums[...] / jnp.maximum(cnts[...], 1.0)
    r = jnp.dot(pooled, w[...], preferred_element_type=_f32) + b[...]
    out_ref[...] = r[:NGRAPH]


_final_call = pl.pallas_call(
    _final_body,
    grid=(1,),
    in_specs=[
        pl.BlockSpec((GP, HH), lambda i: (0, 0)),
        pl.BlockSpec((GP, HH), lambda i: (0, 0)),
        pl.BlockSpec((HH, HH), lambda i: (0, 0)),
        pl.BlockSpec((1, HH), lambda i: (0, 0)),
    ],
    out_specs=pl.BlockSpec((NGRAPH, HH), lambda i: (0, 0)),
    out_shape=_S((NGRAPH, HH)),
)


def kernel(x, edge_index, batch, W_in, b_in, W1_0, b1_0, W2_0, b2_0,
           W1_1, b1_1, W2_1, b2_1, W_lin, b_lin):
    src = edge_index[0]
    dst = edge_index[1]
    npad = E2 - EE
    padG = jnp.arange(npad, dtype=jnp.int32) % NN
    padD = NN + (jnp.arange(npad, dtype=jnp.int32) % (AGN - NN))
    srcG = jnp.concatenate([src, padG]).reshape(NW, NCH, 1, C)
    srcD = jnp.concatenate([src, padD]).reshape(NW, NCH, 1, C)
    dstP = jnp.concatenate([dst, padD]).reshape(NW, NCH, 1, C)
    ones128 = jnp.ones((NN, HH), _f32)
    zeros128 = jnp.zeros((RPT, HH), _f32)
    batch2 = batch.reshape(NN, 1)

    deg0, deg1 = _scatter_kernel(ones128, srcG, srcD, zeros128)
    h, hd, dinv16, counts = _init_call(
        x, W_in, b_in.reshape(1, HH), deg0, deg1, batch2)

    sums = None
    for layer, (W1, b1, W2, b2) in enumerate(
            ((W1_0, b1_0, W2_0, b2_0), (W1_1, b1_1, W2_1, b2_1))):
        powers = [h]
        step = 0
        for tgt in SCALES:
            while step < tgt:
                a0, a1 = _scatter_kernel(hd, srcG, dstP, zeros128)
                h, hd = _update_call(h, a0, a1, dinv16)
                step += 1
            powers.append(h)
        w1r = W1.reshape(6, HH, HH)
        if layer == 0:
            h, hd = _mlp1_call(*powers, w1r, b1.reshape(1, HH),
                               W2, b2.reshape(1, HH), dinv16)
        else:
            sums = _mlp2_call(*powers, w1r, b1.reshape(1, HH),
                              W2, b2.reshape(1, HH), batch2)

    return _final_call(sums, counts, W_lin, b_lin.reshape(1, HH))


# speedup vs baseline: 3.4292x; 1.0123x over previous
import functools

import jax
import jax.numpy as jnp
from jax import lax
from jax.experimental import pallas as pl
from jax.experimental.pallas import tpu as pltpu
from jax.experimental.pallas import tpu_sc as plsc

NN = 10000
EE = 320000
HH = 128
NGRAPH = 100
GP = 104
SCALES = [1, 2, 4, 8, 16, 32]

NC = 2
NS = 16
NW = NC * NS
C = 92
NBUF = 4
NCH = 112
EPT = NCH * C
E2 = NW * EPT
NGRP = NCH // NBUF
AGN = 10104
RPT = 632
RPT_LAST = NN - 15 * RPT

_f32 = jnp.float32
_mesh = plsc.VectorSubcoreMesh(
    core_axis_name="c", subcore_axis_name="s", num_cores=NC)


def _S(shape):
    return jax.ShapeDtypeStruct(shape, _f32)


def _tile_rows(body_fn, s):
    row0 = pl.multiple_of(s * RPT, 8)

    @pl.when(s < 15)
    def _():
        body_fn(row0, RPT)

    @pl.when(s == 15)
    def _():
        body_fn(row0, RPT_LAST)


@functools.partial(
    pl.kernel,
    out_type=(_S((NN, HH)), _S((NN, HH))),
    mesh=_mesh,
    scratch_types=[
        pltpu.VMEM((NBUF, 1, C), jnp.int32),
        pltpu.VMEM((NBUF, 1, C), jnp.int32),
        pltpu.VMEM((NBUF, C, HH), _f32),
        pltpu.VMEM_SHARED((AGN, HH), _f32),
        pltpu.SemaphoreType.DMA((NBUF,)),
        pltpu.SemaphoreType.DMA((NBUF,)),
        pltpu.SemaphoreType.DMA((NBUF,)),
        pltpu.SemaphoreType.DMA((NBUF,)),
        pltpu.SemaphoreType.DMA,
    ],
)
def _scatter_kernel(hd_hbm, src_hbm, dst_hbm, zeros_hbm, agg0_hbm, agg1_hbm,
                    isrc_v, idst_v, rows_v, agg_sh, isem, dsem, gsem, ssem,
                    zsem):
    cc = lax.axis_index("c")
    s = lax.axis_index("s")
    w = cc * NS + s

    def zero_issue(row0, nrows):
        pltpu.async_copy(zeros_hbm.at[pl.ds(0, nrows)],
                        agg_sh.at[pl.ds(row0, nrows)], zsem)

    def zero_wait(row0, nrows):
        pltpu.make_async_copy(zeros_hbm.at[pl.ds(0, nrows)],
                              agg_sh.at[pl.ds(row0, nrows)], zsem).wait()

    _tile_rows(zero_issue, s)

    @pl.when(s == 15)
    def _():
        pltpu.async_copy(zeros_hbm.at[pl.ds(0, AGN - NN)],
                         agg_sh.at[pl.ds(NN, AGN - NN)], zsem)

    for b in range(NBUF):
        pltpu.async_copy(src_hbm.at[w, b], isrc_v.at[b], isem.at[b])

    _tile_rows(zero_wait, s)

    @pl.when(s == 15)
    def _():
        pltpu.make_async_copy(zeros_hbm.at[pl.ds(0, AGN - NN)],
                              agg_sh.at[pl.ds(NN, AGN - NN)], zsem).wait()

    plsc.subcore_barrier()

    def grp(g, carry):
        for b in range(NBUF):
            j = g * NBUF + b

            @pl.when(g > 0)
            def _():
                pltpu.make_async_copy(
                    rows_v.at[b], agg_sh.at[idst_v.at[b, 0]],
                    ssem.at[b]).wait()

            pltpu.async_copy(dst_hbm.at[w, j], idst_v.at[b], dsem.at[b])
            pltpu.make_async_copy(
                src_hbm.at[w, j], isrc_v.at[b], isem.at[b]).wait()
            pltpu.async_copy(
                hd_hbm.at[isrc_v.at[b, 0]], rows_v.at[b], gsem.at[b])
        for b in range(NBUF):
            j = g * NBUF + b
            pltpu.make_async_copy(
                hd_hbm.at[isrc_v.at[b, 0]], rows_v.at[b], gsem.at[b]).wait()

            @pl.when(g < NGRP - 1)
            def _():
                pltpu.async_copy(
                    src_hbm.at[w, j + NBUF], isrc_v.at[b], isem.at[b])

            pltpu.make_async_copy(
                dst_hbm.at[w, j], idst_v.at[b], dsem.at[b]).wait()
            pltpu.async_copy(
                rows_v.at[b], agg_sh.at[idst_v.at[b, 0]], ssem.at[b],
                add=True)
        return carry

    lax.fori_loop(0, NGRP, grp, 0)
    for b in range(NBUF):
        pltpu.make_async_copy(
            rows_v.at[b], agg_sh.at[idst_v.at[b, 0]], ssem.at[b]).wait()
    plsc.subcore_barrier()

    def export0(row0, nrows):
        pltpu.sync_copy(agg_sh.at[pl.ds(row0, nrows)],
                        agg0_hbm.at[pl.ds(row0, nrows)])

    def export1(row0, nrows):
        pltpu.sync_copy(agg_sh.at[pl.ds(row0, nrows)],
                        agg1_hbm.at[pl.ds(row0, nrows)])

    @pl.when(cc == 0)
    def _():
        _tile_rows(export0, s)

    @pl.when(cc == 1)
    def _():
        _tile_rows(export1, s)


RB = 400
RBU = 1000


def _init_body(x, w, b, d0, d1, bt, h_ref, hd_ref, dv_ref, cnt_ref):
    i = pl.program_id(0)
    h = jnp.dot(x[...], w[...], preferred_element_type=_f32) + b[...]
    dinv = 1.0 / jnp.maximum(d0[...][:, :1] + d1[...][:, :1], 1.0)
    h_ref[...] = h
    hd_ref[...] = h * dinv
    dv_ref[...] = dinv * jnp.ones((1, 16), _f32)
    oh = (bt[...] == lax.broadcasted_iota(jnp.int32, (1, GP), 1)).astype(_f32)
    cnt = jnp.sum(oh, axis=0)[:, None] * jnp.ones((1, HH), _f32)

    @pl.when(i == 0)
    def _():
        cnt_ref[...] = cnt

    @pl.when(i != 0)
    def _():
        cnt_ref[...] += cnt


_init_call = pl.pallas_call(
    _init_body,
    grid=(NN // RB,),
    in_specs=[
        pl.BlockSpec((RB, HH), lambda i: (i, 0)),
        pl.BlockSpec((HH, HH), lambda i: (0, 0)),
        pl.BlockSpec((1, HH), lambda i: (0, 0)),
        pl.BlockSpec((RB, HH), lambda i: (i, 0)),
        pl.BlockSpec((RB, HH), lambda i: (i, 0)),
        pl.BlockSpec((RB, 1), lambda i: (i, 0)),
    ],
    out_specs=[
        pl.BlockSpec((RB, HH), lambda i: (i, 0)),
        pl.BlockSpec((RB, HH), lambda i: (i, 0)),
        pl.BlockSpec((RB, 16), lambda i: (i, 0)),
        pl.BlockSpec((GP, HH), lambda i: (0, 0)),
    ],
    out_shape=[_S((NN, HH)), _S((NN, HH)), _S((NN, 16)), _S((GP, HH))],
)


def _update_body(h, a0, a1, dv, hn_ref, hdn_ref):
    hn = 0.5 * (h[...] + (a0[...] + a1[...]))
    hn_ref[...] = hn
    hdn_ref[...] = hn * dv[...][:, :1]


_update_call = pl.pallas_call(
    _update_body,
    grid=(NN // RBU,),
    in_specs=[
        pl.BlockSpec((RBU, HH), lambda i: (i, 0)),
        pl.BlockSpec((RBU, HH), lambda i: (i, 0)),
        pl.BlockSpec((RBU, HH), lambda i: (i, 0)),
        pl.BlockSpec((RBU, 16), lambda i: (i, 0)),
    ],
    out_specs=[
        pl.BlockSpec((RBU, HH), lambda i: (i, 0)),
        pl.BlockSpec((RBU, HH), lambda i: (i, 0)),
    ],
    out_shape=[_S((NN, HH)), _S((NN, HH))],
)


def _mlp_core(ps, w1, b1, w2, b2):
    acc = b1
    for j in range(6):
        wav = jnp.maximum(ps[j] - ps[j + 1], 0.0)
        acc = acc + jnp.dot(wav, w1[j], preferred_element_type=_f32)
    z = jnp.maximum(acc, 0.0)
    return jnp.dot(z, w2, preferred_element_type=_f32) + b2


def _mlp1_body(p0, p1, p2, p3, p4, p5, p6, w1, b1, w2, b2, dv,
               h_ref, hd_ref):
    ps = [p0[...], p1[...], p2[...], p3[...], p4[...], p5[...], p6[...]]
    out = _mlp_core(ps, w1[...], b1[...], w2[...], b2[...])
    h_ref[...] = out
    hd_ref[...] = out * dv[...][:, :1]


_p_spec = pl.BlockSpec((RB, HH), lambda i: (i, 0))
_w_specs = [
    pl.BlockSpec((6, HH, HH), lambda i: (0, 0, 0)),
    pl.BlockSpec((1, HH), lambda i: (0, 0)),
    pl.BlockSpec((HH, HH), lambda i: (0, 0)),
    pl.BlockSpec((1, HH), lambda i: (0, 0)),
]

_mlp1_call = pl.pallas_call(
    _mlp1_body,
    grid=(NN // RB,),
    in_specs=[_p_spec] * 7 + _w_specs + [pl.BlockSpec((RB, 16), lambda i: (i, 0))],
    out_specs=[_p_spec, _p_spec],
    out_shape=[_S((NN, HH)), _S((NN, HH))],
)


def _mlp2_body(p0, p1, p2, p3, p4, p5, p6, w1, b1, w2, b2, bt, sum_ref):
    i = pl.program_id(0)
    ps = [p0[...], p1[...], p2[...], p3[...], p4[...], p5[...], p6[...]]
    out = _mlp_core(ps, w1[...], b1[...], w2[...], b2[...])
    oh = (bt[...] == lax.broadcasted_iota(jnp.int32, (1, GP), 1)).astype(_f32)
    part = lax.dot_general(oh, out, (((0,), (0,)), ((), ())),
                           preferred_element_type=_f32)

    @pl.when(i == 0)
    def _():
        sum_ref[...] = part

    @pl.when(i != 0)
    def _():
        sum_ref[...] += part


_mlp2_call = pl.pallas_call(
    _mlp2_body,
    grid=(NN // RB,),
    in_specs=[_p_spec] * 7 + _w_specs + [pl.BlockSpec((RB, 1), lambda i: (i, 0))],
    out_specs=pl.BlockSpec((GP, HH), lambda i: (0, 0)),
    out_shape=_S((GP, HH)),
)


def _final_body(sums, cnts, w, b, out_ref):
    pooled = sums[...] / jnp.maximum(cnts[...], 1.0)
    r = jnp.dot(pooled, w[...], preferred_element_type=_f32) + b[...]
    out_ref[...] = r[:NGRAPH]


_final_call = pl.pallas_call(
    _final_body,
    grid=(1,),
    in_specs=[
        pl.BlockSpec((GP, HH), lambda i: (0, 0)),
        pl.BlockSpec((GP, HH), lambda i: (0, 0)),
        pl.BlockSpec((HH, HH), lambda i: (0, 0)),
        pl.BlockSpec((1, HH), lambda i: (0, 0)),
    ],
    out_specs=pl.BlockSpec((NGRAPH, HH), lambda i: (0, 0)),
    out_shape=_S((NGRAPH, HH)),
)


def kernel(x, edge_index, batch, W_in, b_in, W1_0, b1_0, W2_0, b2_0,
           W1_1, b1_1, W2_1, b2_1, W_lin, b_lin):
    src = edge_index[0]
    dst = edge_index[1]
    npad = E2 - EE
    padG = jnp.arange(npad, dtype=jnp.int32) % NN
    padD = NN + (jnp.arange(npad, dtype=jnp.int32) % (AGN - NN))
    srcG = jnp.concatenate([src, padG]).reshape(NW, NCH, 1, C)
    srcD = jnp.concatenate([src, padD]).reshape(NW, NCH, 1, C)
    dstP = jnp.concatenate([dst, padD]).reshape(NW, NCH, 1, C)
    ones128 = jnp.ones((NN, HH), _f32)
    zeros128 = jnp.zeros((RPT, HH), _f32)
    batch2 = batch.reshape(NN, 1)

    deg0, deg1 = _scatter_kernel(ones128, srcG, srcD, zeros128)
    h, hd, dinv16, counts = _init_call(
        x, W_in, b_in.reshape(1, HH), deg0, deg1, batch2)

    sums = None
    for layer, (W1, b1, W2, b2) in enumerate(
            ((W1_0, b1_0, W2_0, b2_0), (W1_1, b1_1, W2_1, b2_1))):
        powers = [h]
        step = 0
        for tgt in SCALES:
            while step < tgt:
                a0, a1 = _scatter_kernel(hd, srcG, dstP, zeros128)
                h, hd = _update_call(h, a0, a1, dinv16)
                step += 1
            powers.append(h)
        w1r = W1.reshape(6, HH, HH)
        if layer == 0:
            h, hd = _mlp1_call(*powers, w1r, b1.reshape(1, HH),
                               W2, b2.reshape(1, HH), dinv16)
        else:
            sums = _mlp2_call(*powers, w1r, b1.reshape(1, HH),
                              W2, b2.reshape(1, HH), batch2)

    return _final_call(sums, counts, W_lin, b_lin.reshape(1, HH))


# speedup vs baseline: 3.4455x; 1.0048x over previous
import functools

import jax
import jax.numpy as jnp
from jax import lax
from jax.experimental import pallas as pl
from jax.experimental.pallas import tpu as pltpu
from jax.experimental.pallas import tpu_sc as plsc

NN = 10000
EE = 320000
HH = 128
NGRAPH = 100
GP = 104
SCALES = [1, 2, 4, 8, 16, 32]

NC = 2
NS = 16
NW = NC * NS
C = 92
NBUF = 4
NCH = 112
EPT = NCH * C
E2 = NW * EPT
NGRP = NCH // NBUF
AGN = 10104
RPT = 632
RPT_LAST = NN - 15 * RPT

_f32 = jnp.float32
_mesh = plsc.VectorSubcoreMesh(
    core_axis_name="c", subcore_axis_name="s", num_cores=NC)


def _S(shape):
    return jax.ShapeDtypeStruct(shape, _f32)


def _tile_rows(body_fn, s):
    row0 = pl.multiple_of(s * RPT, 8)

    @pl.when(s < 15)
    def _():
        body_fn(row0, RPT)

    @pl.when(s == 15)
    def _():
        body_fn(row0, RPT_LAST)


@functools.partial(
    pl.kernel,
    out_type=(_S((NN, HH)), _S((NN, HH))),
    mesh=_mesh,
    scratch_types=[
        pltpu.VMEM((NBUF, 1, C), jnp.int32),
        pltpu.VMEM((NBUF, 1, C), jnp.int32),
        pltpu.VMEM((NBUF, C, HH), _f32),
        pltpu.VMEM_SHARED((AGN, HH), _f32),
        pltpu.SemaphoreType.DMA((NBUF,)),
        pltpu.SemaphoreType.DMA((NBUF,)),
        pltpu.SemaphoreType.DMA((NBUF,)),
        pltpu.SemaphoreType.DMA((NBUF,)),
        pltpu.SemaphoreType.DMA,
    ],
)
def _scatter_kernel(hd_hbm, src_hbm, dst_hbm, zeros_hbm, agg0_hbm, agg1_hbm,
                    isrc_v, idst_v, rows_v, agg_sh, isem, dsem, gsem, ssem,
                    zsem):
    cc = lax.axis_index("c")
    s = lax.axis_index("s")
    w = cc * NS + s

    def zero_issue(row0, nrows):
        pltpu.async_copy(zeros_hbm.at[pl.ds(0, nrows)],
                        agg_sh.at[pl.ds(row0, nrows)], zsem)

    def zero_wait(row0, nrows):
        pltpu.make_async_copy(zeros_hbm.at[pl.ds(0, nrows)],
                              agg_sh.at[pl.ds(row0, nrows)], zsem).wait()

    _tile_rows(zero_issue, s)

    @pl.when(s == 15)
    def _():
        pltpu.async_copy(zeros_hbm.at[pl.ds(0, AGN - NN)],
                         agg_sh.at[pl.ds(NN, AGN - NN)], zsem)

    for b in range(NBUF):
        pltpu.async_copy(src_hbm.at[w, b], isrc_v.at[b], isem.at[b])

    _tile_rows(zero_wait, s)

    @pl.when(s == 15)
    def _():
        pltpu.make_async_copy(zeros_hbm.at[pl.ds(0, AGN - NN)],
                              agg_sh.at[pl.ds(NN, AGN - NN)], zsem).wait()

    plsc.subcore_barrier()

    def grp(g, carry):
        for b in range(NBUF):
            j = g * NBUF + b

            @pl.when(g > 0)
            def _():
                pltpu.make_async_copy(
                    rows_v.at[b], agg_sh.at[idst_v.at[b, 0]],
                    ssem.at[b]).wait()

            pltpu.async_copy(dst_hbm.at[w, j], idst_v.at[b], dsem.at[b])
            pltpu.make_async_copy(
                src_hbm.at[w, j], isrc_v.at[b], isem.at[b]).wait()
            pltpu.async_copy(
                hd_hbm.at[isrc_v.at[b, 0]], rows_v.at[b], gsem.at[b])
        for b in range(NBUF):
            j = g * NBUF + b
            pltpu.make_async_copy(
                hd_hbm.at[isrc_v.at[b, 0]], rows_v.at[b], gsem.at[b]).wait()

            @pl.when(g < NGRP - 1)
            def _():
                pltpu.async_copy(
                    src_hbm.at[w, j + NBUF], isrc_v.at[b], isem.at[b])

            pltpu.make_async_copy(
                dst_hbm.at[w, j], idst_v.at[b], dsem.at[b]).wait()
            pltpu.async_copy(
                rows_v.at[b], agg_sh.at[idst_v.at[b, 0]], ssem.at[b],
                add=True)
        return carry

    lax.fori_loop(0, NGRP, grp, 0)
    for b in range(NBUF):
        pltpu.make_async_copy(
            rows_v.at[b], agg_sh.at[idst_v.at[b, 0]], ssem.at[b]).wait()
    plsc.subcore_barrier()

    def export0(row0, nrows):
        pltpu.sync_copy(agg_sh.at[pl.ds(row0, nrows)],
                        agg0_hbm.at[pl.ds(row0, nrows)])

    def export1(row0, nrows):
        pltpu.sync_copy(agg_sh.at[pl.ds(row0, nrows)],
                        agg1_hbm.at[pl.ds(row0, nrows)])

    @pl.when(cc == 0)
    def _():
        _tile_rows(export0, s)

    @pl.when(cc == 1)
    def _():
        _tile_rows(export1, s)


RB = 400
RBU = 1000


def _init_body(x, w, b, d0, d1, bt, hd_ref, dv_ref, cnt_ref):
    i = pl.program_id(0)
    h = jnp.dot(x[...], w[...], preferred_element_type=_f32) + b[...]
    dinv = 1.0 / jnp.maximum(d0[...][:, :1] + d1[...][:, :1], 1.0)
    hd_ref[...] = h * dinv
    dv_ref[...] = dinv * jnp.ones((1, 16), _f32)
    oh = (bt[...] == lax.broadcasted_iota(jnp.int32, (1, GP), 1)).astype(_f32)
    cnt = jnp.sum(oh, axis=0)[:, None] * jnp.ones((1, HH), _f32)

    @pl.when(i == 0)
    def _():
        cnt_ref[...] = cnt

    @pl.when(i != 0)
    def _():
        cnt_ref[...] += cnt


_init_call = pl.pallas_call(
    _init_body,
    grid=(NN // RB,),
    in_specs=[
        pl.BlockSpec((RB, HH), lambda i: (i, 0)),
        pl.BlockSpec((HH, HH), lambda i: (0, 0)),
        pl.BlockSpec((1, HH), lambda i: (0, 0)),
        pl.BlockSpec((RB, HH), lambda i: (i, 0)),
        pl.BlockSpec((RB, HH), lambda i: (i, 0)),
        pl.BlockSpec((RB, 1), lambda i: (i, 0)),
    ],
    out_specs=[
        pl.BlockSpec((RB, HH), lambda i: (i, 0)),
        pl.BlockSpec((RB, 16), lambda i: (i, 0)),
        pl.BlockSpec((GP, HH), lambda i: (0, 0)),
    ],
    out_shape=[_S((NN, HH)), _S((NN, 16)), _S((GP, HH))],
)


def _update_body(hd, a0, a1, dv, hdn_ref):
    hdn_ref[...] = 0.5 * (hd[...] + (a0[...] + a1[...]) * dv[...][:, :1])


_update_call = pl.pallas_call(
    _update_body,
    grid=(NN // RBU,),
    in_specs=[
        pl.BlockSpec((RBU, HH), lambda i: (i, 0)),
        pl.BlockSpec((RBU, HH), lambda i: (i, 0)),
        pl.BlockSpec((RBU, HH), lambda i: (i, 0)),
        pl.BlockSpec((RBU, 16), lambda i: (i, 0)),
    ],
    out_specs=pl.BlockSpec((RBU, HH), lambda i: (i, 0)),
    out_shape=_S((NN, HH)),
)


def _mlp_core(ps, md, w1, b1, w2, b2):
    acc = b1
    for j in range(6):
        wav = jnp.maximum((ps[j] - ps[j + 1]) * md, 0.0)
        acc = acc + jnp.dot(wav, w1[j], preferred_element_type=_f32)
    z = jnp.maximum(acc, 0.0)
    return jnp.dot(z, w2, preferred_element_type=_f32) + b2


def _mlp1_body(p0, p1, p2, p3, p4, p5, p6, w1, b1, w2, b2, dv, hd_ref):
    ps = [p0[...], p1[...], p2[...], p3[...], p4[...], p5[...], p6[...]]
    dinv = dv[...][:, :1]
    out = _mlp_core(ps, 1.0 / dinv, w1[...], b1[...], w2[...], b2[...])
    hd_ref[...] = out * dinv


_p_spec = pl.BlockSpec((RB, HH), lambda i: (i, 0))
_w_specs = [
    pl.BlockSpec((6, HH, HH), lambda i: (0, 0, 0)),
    pl.BlockSpec((1, HH), lambda i: (0, 0)),
    pl.BlockSpec((HH, HH), lambda i: (0, 0)),
    pl.BlockSpec((1, HH), lambda i: (0, 0)),
]

_mlp1_call = pl.pallas_call(
    _mlp1_body,
    grid=(NN // RB,),
    in_specs=[_p_spec] * 7 + _w_specs + [pl.BlockSpec((RB, 16), lambda i: (i, 0))],
    out_specs=_p_spec,
    out_shape=_S((NN, HH)),
)


def _mlp2_body(p0, p1, p2, p3, p4, p5, p6, w1, b1, w2, b2, dv, bt, sum_ref):
    i = pl.program_id(0)
    ps = [p0[...], p1[...], p2[...], p3[...], p4[...], p5[...], p6[...]]
    md = 1.0 / dv[...][:, :1]
    out = _mlp_core(ps, md, w1[...], b1[...], w2[...], b2[...])
    oh = (bt[...] == lax.broadcasted_iota(jnp.int32, (1, GP), 1)).astype(_f32)
    part = lax.dot_general(oh, out, (((0,), (0,)), ((), ())),
                           preferred_element_type=_f32)

    @pl.when(i == 0)
    def _():
        sum_ref[...] = part

    @pl.when(i != 0)
    def _():
        sum_ref[...] += part


_mlp2_call = pl.pallas_call(
    _mlp2_body,
    grid=(NN // RB,),
    in_specs=[_p_spec] * 7 + _w_specs + [
        pl.BlockSpec((RB, 16), lambda i: (i, 0)),
        pl.BlockSpec((RB, 1), lambda i: (i, 0))],
    out_specs=pl.BlockSpec((GP, HH), lambda i: (0, 0)),
    out_shape=_S((GP, HH)),
)


def _final_body(sums, cnts, w, b, out_ref):
    pooled = sums[...] / jnp.maximum(cnts[...], 1.0)
    r = jnp.dot(pooled, w[...], preferred_element_type=_f32) + b[...]
    out_ref[...] = r[:NGRAPH]


_final_call = pl.pallas_call(
    _final_body,
    grid=(1,),
    in_specs=[
        pl.BlockSpec((GP, HH), lambda i: (0, 0)),
        pl.BlockSpec((GP, HH), lambda i: (0, 0)),
        pl.BlockSpec((HH, HH), lambda i: (0, 0)),
        pl.BlockSpec((1, HH), lambda i: (0, 0)),
    ],
    out_specs=pl.BlockSpec((NGRAPH, HH), lambda i: (0, 0)),
    out_shape=_S((NGRAPH, HH)),
)


def kernel(x, edge_index, batch, W_in, b_in, W1_0, b1_0, W2_0, b2_0,
           W1_1, b1_1, W2_1, b2_1, W_lin, b_lin):
    src = edge_index[0]
    dst = edge_index[1]
    npad = E2 - EE
    padG = jnp.arange(npad, dtype=jnp.int32) % NN
    padD = NN + (jnp.arange(npad, dtype=jnp.int32) % (AGN - NN))
    srcG = jnp.concatenate([src, padG]).reshape(NW, NCH, 1, C)
    srcD = jnp.concatenate([src, padD]).reshape(NW, NCH, 1, C)
    dstP = jnp.concatenate([dst, padD]).reshape(NW, NCH, 1, C)
    ones128 = jnp.ones((NN, HH), _f32)
    zeros128 = jnp.zeros((RPT, HH), _f32)
    batch2 = batch.reshape(NN, 1)

    deg0, deg1 = _scatter_kernel(ones128, srcG, srcD, zeros128)
    hd, dinv16, counts = _init_call(
        x, W_in, b_in.reshape(1, HH), deg0, deg1, batch2)

    sums = None
    for layer, (W1, b1, W2, b2) in enumerate(
            ((W1_0, b1_0, W2_0, b2_0), (W1_1, b1_1, W2_1, b2_1))):
        powers = [hd]
        step = 0
        for tgt in SCALES:
            while step < tgt:
                a0, a1 = _scatter_kernel(hd, srcG, dstP, zeros128)
                hd = _update_call(hd, a0, a1, dinv16)
                step += 1
            powers.append(hd)
        w1r = W1.reshape(6, HH, HH)
        if layer == 0:
            hd = _mlp1_call(*powers, w1r, b1.reshape(1, HH),
                            W2, b2.reshape(1, HH), dinv16)
        else:
            sums = _mlp2_call(*powers, w1r, b1.reshape(1, HH),
                              W2, b2.reshape(1, HH), dinv16, batch2)

    return _final_call(sums, counts, W_lin, b_lin.reshape(1, HH))
